# Initial kernel scaffold; baseline (speedup 1.0000x reference)
import jax
import jax.numpy as jnp
from jax.experimental import pallas as pl


def kernel(input_ids, word_emb, pos_emb, type_emb, conv_w, ln_gamma, ln_beta):
    raise NotImplementedError("write your pallas kernel here")


# speedup vs baseline: 1.4650x; 1.4650x over previous
import functools

import jax
import jax.numpy as jnp
from jax import lax
from jax.experimental import pallas as pl
from jax.experimental.pallas import tpu as pltpu
from jax.experimental.pallas import tpu_sc as plsc

EPS = 1e-12
L = 16
CHUNK = 128


def _sum_lanes(v):
    dnums = lax.GatherDimensionNumbers(
        offset_dims=(), collapsed_slice_dims=(0,), start_index_map=(0,))
    lanes = lax.iota(jnp.int32, L)
    for d in (1, 2, 4, 8):
        perm = (lanes ^ d).reshape(L, 1)
        v = v + lax.gather(v, perm, dimension_numbers=dnums, slice_sizes=(1,),
                           mode=lax.GatherScatterMode.PROMISE_IN_BOUNDS)
    return v


def _rsqrt16(v):
    bits = lax.bitcast_convert_type(v, jnp.int32)
    y = lax.bitcast_convert_type(
        jnp.int32(0x5F3759DF) - lax.shift_right_logical(bits, 1), jnp.float32)
    for _ in range(3):
        y = y * (1.5 - 0.5 * v * y * y)
    return y


def _make_sc_kernel(N, V, H, S):
    info = plsc.get_sparse_core_info()
    NC, NS = info.num_cores, info.num_subcores
    NW = NC * NS
    assert N % (NW * CHUNK) == 0
    tw = N // NW
    nchunks = tw // CHUNK

    mesh = plsc.VectorSubcoreMesh(core_axis_name="c", subcore_axis_name="s")

    @functools.partial(
        pl.kernel,
        mesh=mesh,
        compiler_params=pltpu.CompilerParams(use_tc_tiling_on_sc=False),
        out_type=jax.ShapeDtypeStruct((N, H), jnp.float32),
        scratch_types=[
            pltpu.VMEM((CHUNK,), jnp.int32),
            pltpu.VMEM((CHUNK, H), jnp.float32),
            pltpu.VMEM((CHUNK, H), jnp.float32),
            pltpu.VMEM((S, H), jnp.float32),
            pltpu.VMEM((L,), jnp.float32),
            pltpu.VMEM((H,), jnp.float32),
            pltpu.VMEM((H,), jnp.float32),
            pltpu.SemaphoreType.DMA,
        ],
    )
    def k(ids_hbm, wemb_hbm, atab_hbm, w0_hbm, g_hbm, b_hbm, out_hbm,
          idx_v, rows_v, obuf_v, atab_v, w0_v, g_v, b_v, sem):
        wid = lax.axis_index("s") * NC + lax.axis_index("c")
        base = wid * tw

        pltpu.sync_copy(atab_hbm, atab_v)
        pltpu.sync_copy(w0_hbm, w0_v)
        pltpu.sync_copy(g_hbm, g_v)
        pltpu.sync_copy(b_hbm, b_v)

        w0 = w0_v[...]
        gs = [g_v[pl.ds(i * L, L)] for i in range(H // L)]
        bs = [b_v[pl.ds(i * L, L)] for i in range(H // L)]

        def chunk_body(c, _):
            t0 = base + c * CHUNK
            pltpu.sync_copy(ids_hbm.at[pl.ds(t0, CHUNK)], idx_v)
            pltpu.async_copy(wemb_hbm.at[idx_v], rows_v, sem).wait()

            def tok_body(t, _):
                s = lax.rem(t0 + t, S)
                xs = []
                for i in range(H // L):
                    v = rows_v[t, pl.ds(i * L, L)]
                    a = atab_v[s, pl.ds(i * L, L)]
                    xs.append(v * w0 + a)
                sv = (xs[0] + xs[1]) + (xs[2] + xs[3])
                qv = (xs[0] * xs[0] + xs[1] * xs[1]) + \
                     (xs[2] * xs[2] + xs[3] * xs[3])
                meanv = _sum_lanes(sv) * (1.0 / H)
                msqv = _sum_lanes(qv) * (1.0 / H)
                varv = msqv - meanv * meanv
                inv = _rsqrt16(varv + EPS)
                for i in range(H // L):
                    o = (xs[i] - meanv) * (inv * gs[i]) + bs[i]
                    obuf_v[t, pl.ds(i * L, L)] = o
                return _

            lax.fori_loop(0, CHUNK, tok_body, None)
            pltpu.sync_copy(obuf_v, out_hbm.at[pl.ds(t0, CHUNK)])
            return _

        lax.fori_loop(0, nchunks, chunk_body, None)

    return k


def kernel(input_ids, word_emb, pos_emb, type_emb, conv_w, ln_gamma, ln_beta):
    B, S = input_ids.shape
    V, H = word_emb.shape
    w = conv_w.reshape(2).astype(jnp.float32)
    atab = w[1] * pos_emb[:S] + type_emb[0]
    w0v = jnp.full((L,), w[0], jnp.float32)
    ids = input_ids.reshape(-1).astype(jnp.int32)
    N = B * S
    out = _make_sc_kernel(N, V, H, S)(
        ids, word_emb, atab, w0v,
        ln_gamma.astype(jnp.float32), ln_beta.astype(jnp.float32))
    return out.reshape(B, S, H)


# speedup vs baseline: 1.9488x; 1.3302x over previous
import functools

import jax
import jax.numpy as jnp
from jax import lax
from jax.experimental import pallas as pl
from jax.experimental.pallas import tpu as pltpu
from jax.experimental.pallas import tpu_sc as plsc

EPS = 1e-12
L = 16
CHUNK = 128


def _sum_lanes(v):
    dnums = lax.GatherDimensionNumbers(
        offset_dims=(), collapsed_slice_dims=(0,), start_index_map=(0,))
    lanes = lax.iota(jnp.int32, L)
    for d in (1, 2, 4, 8):
        perm = (lanes ^ d).reshape(L, 1)
        v = v + lax.gather(v, perm, dimension_numbers=dnums, slice_sizes=(1,),
                           mode=lax.GatherScatterMode.PROMISE_IN_BOUNDS)
    return v


def _rsqrt16(v):
    bits = lax.bitcast_convert_type(v, jnp.int32)
    y = lax.bitcast_convert_type(
        jnp.int32(0x5F3759DF) - lax.shift_right_logical(bits, 1), jnp.float32)
    for _ in range(3):
        y = y * (1.5 - 0.5 * v * y * y)
    return y


def _make_sc_kernel(N, V, H, S):
    info = plsc.get_sparse_core_info()
    NC, NS = info.num_cores, info.num_subcores
    NW = NC * NS
    assert N % (NW * CHUNK) == 0
    tw = N // NW
    nchunks = tw // CHUNK

    mesh = plsc.VectorSubcoreMesh(core_axis_name="c", subcore_axis_name="s")

    @functools.partial(
        pl.kernel,
        mesh=mesh,
        compiler_params=pltpu.CompilerParams(use_tc_tiling_on_sc=False),
        out_type=jax.ShapeDtypeStruct((N, H), jnp.float32),
        scratch_types=[
            pltpu.VMEM((CHUNK,), jnp.int32),
            pltpu.VMEM((CHUNK, H), jnp.float32),
            pltpu.VMEM((CHUNK, H), jnp.float32),
            pltpu.VMEM((S, H), jnp.float32),
            pltpu.VMEM((L,), jnp.float32),
            pltpu.VMEM((H,), jnp.float32),
            pltpu.VMEM((H,), jnp.float32),
            pltpu.SemaphoreType.DMA,
        ],
    )
    def k(ids_hbm, wemb_hbm, atab_hbm, w0_hbm, g_hbm, b_hbm, out_hbm,
          idx_v, rows_v, obuf_v, atab_v, w0_v, g_v, b_v, sem):
        wid = lax.axis_index("s") * NC + lax.axis_index("c")
        base = wid * tw

        pltpu.sync_copy(atab_hbm, atab_v)
        pltpu.sync_copy(w0_hbm, w0_v)
        pltpu.sync_copy(g_hbm, g_v)
        pltpu.sync_copy(b_hbm, b_v)

        w0 = w0_v[...]
        gs = [g_v[pl.ds(i * L, L)] for i in range(H // L)]
        bs = [b_v[pl.ds(i * L, L)] for i in range(H // L)]

        def chunk_body(c, _):
            t0 = base + c * CHUNK
            pltpu.sync_copy(ids_hbm.at[pl.ds(t0, CHUNK)], idx_v)
            pltpu.async_copy(wemb_hbm.at[idx_v], rows_v, sem).wait()

            @plsc.parallel_loop(0, CHUNK, unroll=8)
            def tok_body(t):
                s = lax.rem(t0 + t, S)
                xs = []
                for i in range(H // L):
                    v = rows_v[t, pl.ds(i * L, L)]
                    a = atab_v[s, pl.ds(i * L, L)]
                    xs.append(v * w0 + a)
                sv = (xs[0] + xs[1]) + (xs[2] + xs[3])
                qv = (xs[0] * xs[0] + xs[1] * xs[1]) + \
                     (xs[2] * xs[2] + xs[3] * xs[3])
                meanv = _sum_lanes(sv) * (1.0 / H)
                msqv = _sum_lanes(qv) * (1.0 / H)
                varv = msqv - meanv * meanv
                inv = _rsqrt16(varv + EPS)
                for i in range(H // L):
                    o = (xs[i] - meanv) * (inv * gs[i]) + bs[i]
                    obuf_v[t, pl.ds(i * L, L)] = o

            pltpu.sync_copy(obuf_v, out_hbm.at[pl.ds(t0, CHUNK)])
            return _

        lax.fori_loop(0, nchunks, chunk_body, None)

    return k


def kernel(input_ids, word_emb, pos_emb, type_emb, conv_w, ln_gamma, ln_beta):
    B, S = input_ids.shape
    V, H = word_emb.shape
    w = conv_w.reshape(2).astype(jnp.float32)
    atab = w[1] * pos_emb[:S] + type_emb[0]
    w0v = jnp.full((L,), w[0], jnp.float32)
    ids = input_ids.reshape(-1).astype(jnp.int32)
    N = B * S
    out = _make_sc_kernel(N, V, H, S)(
        ids, word_emb, atab, w0v,
        ln_gamma.astype(jnp.float32), ln_beta.astype(jnp.float32))
    return out.reshape(B, S, H)


# speedup vs baseline: 2.5185x; 1.2923x over previous
import functools

import jax
import jax.numpy as jnp
from jax import lax
from jax.experimental import pallas as pl
from jax.experimental.pallas import tpu as pltpu
from jax.experimental.pallas import tpu_sc as plsc

EPS = 1e-12
L = 16

_DNUMS = lax.GatherDimensionNumbers(
    offset_dims=(), collapsed_slice_dims=(0,), start_index_map=(0,))


def _shuffle(v, idx16):
    return lax.gather(v, idx16.reshape(L, 1), dimension_numbers=_DNUMS,
                      slice_sizes=(1,), mode=lax.GatherScatterMode.PROMISE_IN_BOUNDS)


def _rsqrt16(v):
    bits = lax.bitcast_convert_type(v, jnp.int32)
    y = lax.bitcast_convert_type(
        jnp.int32(0x5F3759DF) - lax.shift_right_logical(bits, 1), jnp.float32)
    for _ in range(3):
        y = y * (1.5 - 0.5 * v * y * y)
    return y


def _make_sc_kernel(B, S, H, V):
    info = plsc.get_sparse_core_info()
    NC, NS = info.num_cores, info.num_subcores
    NW = NC * NS
    BBLK = B // NW
    HB = H // 8
    NG = BBLK // L
    assert B % NW == 0 and BBLK == 128 and H % L == 0 and S % 2 == 0

    mesh = plsc.VectorSubcoreMesh(core_axis_name="c", subcore_axis_name="s")

    @functools.partial(
        pl.kernel,
        mesh=mesh,
        compiler_params=pltpu.CompilerParams(use_tc_tiling_on_sc=False,
                                             needs_layout_passes=False),
        out_type=jax.ShapeDtypeStruct((S, HB, NW, 8, BBLK), jnp.float32),
        scratch_types=[
            pltpu.VMEM((S, BBLK), jnp.int32),
            pltpu.VMEM((BBLK, H), jnp.float32),
            pltpu.VMEM((BBLK, H), jnp.float32),
            pltpu.VMEM((H, BBLK), jnp.float32),
            pltpu.VMEM((H, BBLK), jnp.float32),
            pltpu.VMEM((H * L,), jnp.float32),
            pltpu.VMEM((S, H), jnp.float32),
            pltpu.VMEM((H * L,), jnp.float32),
            pltpu.VMEM((H * L,), jnp.float32),
            pltpu.VMEM((H,), jnp.float32),
            pltpu.VMEM((H,), jnp.float32),
            pltpu.VMEM((L,), jnp.float32),
            pltpu.SemaphoreType.DMA,
            pltpu.SemaphoreType.DMA,
            pltpu.SemaphoreType.DMA,
            pltpu.SemaphoreType.DMA,
        ],
    )
    def k(idsT, wemb, atab_h, w0_h, g_h, b_h, out_h,
          idx_all, rows0, rows1, ob0, ob1, abuf, atab_v, gbc, bbc,
          gtmp, btmp, w0_v, gs0, gs1, ws0, ws1):
        wid = lax.axis_index("s") * NC + lax.axis_index("c")
        b0 = wid * BBLK
        pltpu.sync_copy(idsT.at[:, pl.ds(b0, BBLK)], idx_all)
        pltpu.sync_copy(atab_h, atab_v)
        pltpu.sync_copy(w0_h, w0_v)
        pltpu.sync_copy(g_h, gtmp)
        pltpu.sync_copy(b_h, btmp)

        lanes = lax.iota(jnp.int32, L)
        zero16 = lanes ^ lanes
        for i in range(H // L):
            gv = gtmp[pl.ds(i * L, L)]
            bv = btmp[pl.ds(i * L, L)]
            for j in range(L):
                gbc[pl.ds((i * L + j) * L, L)] = _shuffle(gv, zero16 + j)
                bbc[pl.ds((i * L + j) * L, L)] = _shuffle(bv, zero16 + j)
        w0 = w0_v[...]
        zf = zero16.astype(jnp.float32)
        rowidx = [lanes + lg * L for lg in range(NG)]
        inv_h = 1.0 / H

        rows_bufs = (rows0, rows1)
        ob_bufs = (ob0, ob1)
        gsems = (gs0, gs1)
        wsems = (ws0, ws1)

        def gather_desc(s, par):
            return pltpu.make_async_copy(
                wemb.at[idx_all.at[s]], rows_bufs[par], gsems[par])

        def write_descs(s, par):
            return [pltpu.make_async_copy(
                        ob_bufs[par].at[pl.ds(hb * 8, 8)],
                        out_h.at[s, hb, wid], wsems[par])
                    for hb in range(HB)]

        gather_desc(0, 0).start()

        def process(g, s, par):
            nxt = 1 - par
            if par == 0:
                gather_desc(s + 1, nxt).start()
            else:
                @pl.when(g < S // 2 - 1)
                def _():
                    gather_desc(s + 1, nxt).start()

            @pl.when(g > 0)
            def _():
                for d in write_descs(s, par):
                    d.wait()

            gather_desc(s, par).wait()
            rows_v = rows_bufs[par]
            ob_v = ob_bufs[par]

            for i in range(H // L):
                av = atab_v[s, pl.ds(i * L, L)]
                for j in range(L):
                    abuf[pl.ds((i * L + j) * L, L)] = _shuffle(av, zero16 + j)

            def ph1(h, carry):
                accs = list(carry)
                a_h = abuf[pl.ds(h * L, L)]
                hsplat = jnp.full((L,), h, jnp.int32)
                for lg in range(NG):
                    v = plsc.load_gather(rows_v, [rowidx[lg], hsplat])
                    x = v * w0 + a_h
                    ob_v[h, pl.ds(lg * L, L)] = x
                    accs[2 * lg] = accs[2 * lg] + x
                    accs[2 * lg + 1] = x * x + accs[2 * lg + 1]
                return tuple(accs)

            stats = plsc.parallel_loop(0, H, unroll=2,
                                       carry=tuple([zf] * (2 * NG)))(ph1)

            means, scales = [], []
            for lg in range(NG):
                mean = stats[2 * lg] * inv_h
                var = stats[2 * lg + 1] * inv_h - mean * mean
                means.append(mean)
                scales.append(_rsqrt16(var + EPS))

            def ph3(h):
                gh = gbc[pl.ds(h * L, L)]
                bh = bbc[pl.ds(h * L, L)]
                for lg in range(NG):
                    x = ob_v[h, pl.ds(lg * L, L)]
                    o = (x - means[lg]) * (scales[lg] * gh) + bh
                    ob_v[h, pl.ds(lg * L, L)] = o

            plsc.parallel_loop(0, H, unroll=2)(ph3)

            for d in write_descs(s, par):
                d.start()

        def pair(g, _):
            process(g, 2 * g, 0)
            process(g, 2 * g + 1, 1)
            return _

        lax.fori_loop(0, S // 2, pair, None)

        for par in (0, 1):
            for d in write_descs(0, par):
                d.wait()

    return k


def kernel(input_ids, word_emb, pos_emb, type_emb, conv_w, ln_gamma, ln_beta):
    B, S = input_ids.shape
    V, H = word_emb.shape
    w = conv_w.reshape(2).astype(jnp.float32)
    atab = w[1] * pos_emb[:S] + type_emb[0]
    w0v = jnp.full((L,), w[0], jnp.float32)
    idsT = input_ids.T.astype(jnp.int32)
    out5d = _make_sc_kernel(B, S, H, V)(
        idsT, word_emb, atab, w0v,
        ln_gamma.astype(jnp.float32), ln_beta.astype(jnp.float32))
    NW = out5d.shape[2]
    return jnp.transpose(out5d, (2, 4, 0, 1, 3)).reshape(B, S, H)


# speedup vs baseline: 2.5685x; 1.0199x over previous
import functools

import jax
import jax.numpy as jnp
from jax import lax
from jax.experimental import pallas as pl
from jax.experimental.pallas import tpu as pltpu
from jax.experimental.pallas import tpu_sc as plsc

EPS = 1e-12
L = 16
SPG = 4

_DNUMS = lax.GatherDimensionNumbers(
    offset_dims=(), collapsed_slice_dims=(0,), start_index_map=(0,))


def _shuffle(v, idx16):
    return lax.gather(v, idx16.reshape(L, 1), dimension_numbers=_DNUMS,
                      slice_sizes=(1,), mode=lax.GatherScatterMode.PROMISE_IN_BOUNDS)


def _rsqrt16(v):
    bits = lax.bitcast_convert_type(v, jnp.int32)
    y = lax.bitcast_convert_type(
        jnp.int32(0x5F3759DF) - lax.shift_right_logical(bits, 1), jnp.float32)
    for _ in range(3):
        y = y * (1.5 - 0.5 * v * y * y)
    return y


def _make_sc_kernel(B, S, H, V):
    info = plsc.get_sparse_core_info()
    NC, NS = info.num_cores, info.num_subcores
    NW = NC * NS
    BBLK = B // NW
    HB = H // 8
    NG = BBLK // L
    G = S // SPG
    assert B % NW == 0 and BBLK == 128 and H % L == 0
    assert S % SPG == 0 and G % 2 == 0

    mesh = plsc.VectorSubcoreMesh(core_axis_name="c", subcore_axis_name="s")

    @functools.partial(
        pl.kernel,
        mesh=mesh,
        compiler_params=pltpu.CompilerParams(use_tc_tiling_on_sc=False,
                                             needs_layout_passes=False),
        out_type=jax.ShapeDtypeStruct((S, HB, NW, 8, BBLK), jnp.float32),
        scratch_types=[
            pltpu.VMEM((SPG, BBLK), jnp.int32),
            pltpu.VMEM((SPG, BBLK), jnp.int32),
            pltpu.VMEM((SPG, BBLK, H), jnp.float32),
            pltpu.VMEM((SPG, BBLK, H), jnp.float32),
            pltpu.VMEM((HB, 8, BBLK), jnp.float32),
            pltpu.VMEM((HB, 8, BBLK), jnp.float32),
            pltpu.VMEM((H * L,), jnp.float32),
            pltpu.VMEM((S, H), jnp.float32),
            pltpu.VMEM((H * L,), jnp.float32),
            pltpu.VMEM((H * L,), jnp.float32),
            pltpu.VMEM((H,), jnp.float32),
            pltpu.VMEM((H,), jnp.float32),
            pltpu.VMEM((L,), jnp.float32),
            pltpu.SemaphoreType.DMA,
            pltpu.SemaphoreType.DMA,
            pltpu.SemaphoreType.DMA,
            pltpu.SemaphoreType.DMA,
            pltpu.SemaphoreType.DMA,
            pltpu.SemaphoreType.DMA,
        ],
    )
    def k(idsT, wemb, atab_h, w0_h, g_h, b_h, out_h,
          ib0, ib1, rb0, rb1, ob0, ob1, abuf, atab_v, gbc, bbc,
          gtmp, btmp, w0_v, is0, is1, gs0, gs1, ws0, ws1):
        wid = lax.axis_index("s") * NC + lax.axis_index("c")
        b0 = wid * BBLK
        pltpu.sync_copy(atab_h, atab_v)
        pltpu.sync_copy(w0_h, w0_v)
        pltpu.sync_copy(g_h, gtmp)
        pltpu.sync_copy(b_h, btmp)

        lanes = lax.iota(jnp.int32, L)
        zero16 = lanes ^ lanes
        for i in range(H // L):
            gv = gtmp[pl.ds(i * L, L)]
            bv = btmp[pl.ds(i * L, L)]
            for j in range(L):
                gbc[pl.ds((i * L + j) * L, L)] = _shuffle(gv, zero16 + j)
                bbc[pl.ds((i * L + j) * L, L)] = _shuffle(bv, zero16 + j)
        w0 = w0_v[...]
        zf = zero16.astype(jnp.float32)
        rowidx = [lanes + lg * L for lg in range(NG)]
        inv_h = 1.0 / H

        ibs = (ib0, ib1)
        rbs = (rb0, rb1)
        obs = (ob0, ob1)
        isems = (is0, is1)
        gsems = (gs0, gs1)
        wsems = (ws0, ws1)

        def idx_desc(g, par):
            return pltpu.make_async_copy(
                idsT.at[pl.ds(g * SPG, SPG), pl.ds(b0, BBLK)],
                ibs[par], isems[par])

        def gather_descs(par):
            return [pltpu.make_async_copy(
                        wemb.at[ibs[par].at[sg]], rbs[par].at[sg], gsems[par])
                    for sg in range(SPG)]

        def write_desc(s, par):
            return pltpu.make_async_copy(
                obs[par], out_h.at[s, :, wid], wsems[par])

        pltpu.sync_copy(idsT.at[pl.ds(0, SPG), pl.ds(b0, BBLK)], ib0)
        for d in gather_descs(0):
            d.start()
        idx_desc(1, 1).start()

        def compute_s(s, sg, rows_v, ob_v):
            for i in range(H // L):
                av = atab_v[s, pl.ds(i * L, L)]
                for j in range(L):
                    abuf[pl.ds((i * L + j) * L, L)] = _shuffle(av, zero16 + j)

            sgsplat = zero16 + sg

            def ph1(h, carry):
                accs = list(carry)
                a_h = abuf[pl.ds(h * L, L)]
                hsplat = jnp.full((L,), h, jnp.int32)
                hb = h // 8
                hi = h % 8
                for lg in range(NG):
                    v = plsc.load_gather(rows_v, [sgsplat, rowidx[lg], hsplat])
                    x = v * w0 + a_h
                    ob_v[hb, hi, pl.ds(lg * L, L)] = x
                    accs[2 * lg] = accs[2 * lg] + x
                    accs[2 * lg + 1] = x * x + accs[2 * lg + 1]
                return tuple(accs)

            stats = plsc.parallel_loop(0, H, unroll=2,
                                       carry=tuple([zf] * (2 * NG)))(ph1)

            means, scales = [], []
            for lg in range(NG):
                mean = stats[2 * lg] * inv_h
                var = stats[2 * lg + 1] * inv_h - mean * mean
                means.append(mean)
                scales.append(_rsqrt16(var + EPS))

            def ph3(h):
                gh = gbc[pl.ds(h * L, L)]
                bh = bbc[pl.ds(h * L, L)]
                hb = h // 8
                hi = h % 8
                for lg in range(NG):
                    x = ob_v[hb, hi, pl.ds(lg * L, L)]
                    o = (x - means[lg]) * (scales[lg] * gh) + bh
                    ob_v[hb, hi, pl.ds(lg * L, L)] = o

            plsc.parallel_loop(0, H, unroll=2)(ph3)

        def step(gg, g, par):
            nxt = 1 - par

            def fire_next_gather():
                idx_desc(g + 1, nxt).wait()
                for d in gather_descs(nxt):
                    d.start()

            if par == 0:
                fire_next_gather()
            else:
                pl.when(gg < G // 2 - 1)(fire_next_gather)

            for d in gather_descs(par):
                d.wait()

            @pl.when(gg < G // 2 - 1)
            def _():
                idx_desc(g + 2, par).start()

            rows_v = rbs[par]
            for sg in range(SPG):
                s = g * SPG + sg
                opar = sg % 2
                ob_v = obs[opar]

                @pl.when(s > 1)
                def _():
                    write_desc(s, opar).wait()

                compute_s(s, sg, rows_v, ob_v)
                write_desc(s, opar).start()

        def pair(gg, _):
            step(gg, 2 * gg, 0)
            step(gg, 2 * gg + 1, 1)
            return _

        lax.fori_loop(0, G // 2, pair, None)

        write_desc(0, 0).wait()
        write_desc(1, 1).wait()

    return k


def kernel(input_ids, word_emb, pos_emb, type_emb, conv_w, ln_gamma, ln_beta):
    B, S = input_ids.shape
    V, H = word_emb.shape
    w = conv_w.reshape(2).astype(jnp.float32)
    atab = w[1] * pos_emb[:S] + type_emb[0]
    w0v = jnp.full((L,), w[0], jnp.float32)
    idsT = input_ids.T.astype(jnp.int32)
    out5d = _make_sc_kernel(B, S, H, V)(
        idsT, word_emb, atab, w0v,
        ln_gamma.astype(jnp.float32), ln_beta.astype(jnp.float32))
    return jnp.transpose(out5d, (2, 4, 0, 1, 3)).reshape(B, S, H)
